# Initial kernel scaffold; baseline (speedup 1.0000x reference)
import jax
import jax.numpy as jnp
from jax.experimental import pallas as pl


def kernel(x, edge_index, edge_types, cell_types, max_size, W1, Ws1, b1, W2, Ws2, b2, W3, Ws3, b3):
    raise NotImplementedError("write your pallas kernel here")


# speedup vs baseline: 7.8349x; 7.8349x over previous
import functools

import jax
import jax.numpy as jnp
from jax import lax
from jax.experimental import pallas as pl
from jax.experimental.pallas import tpu as pltpu
from jax.experimental.pallas import tpu_sc as plsc


def _dense_first_body(x_ref, w_ref, ws_ref, b_ref, y_ref, self_ref):
    x = x_ref[...]
    y_ref[0, :, :] = jnp.dot(x, w_ref[0, :, :], preferred_element_type=jnp.float32)
    y_ref[1, :, :] = jnp.dot(x, w_ref[1, :, :], preferred_element_type=jnp.float32)
    self_ref[...] = (
        jnp.dot(x, ws_ref[...], preferred_element_type=jnp.float32) + b_ref[...]
    )


def _dense_first(x, w, ws, b):
    n = x.shape[0]
    h = w.shape[2]
    return pl.pallas_call(
        _dense_first_body,
        out_shape=[
            jax.ShapeDtypeStruct((2, n, h), jnp.float32),
            jax.ShapeDtypeStruct((n, h), jnp.float32),
        ],
    )(x, w, ws, b)


def _make_dense_mid(n):
    def body(p_ref, sp_ref, w_ref, ws_ref, b_ref, y_ref, self_ref):
        hidden = p_ref[0, :n, :] + p_ref[1, :n, :] + sp_ref[...]
        hidden = jnp.maximum(hidden, 0.0)
        y_ref[0, :, :] = jnp.dot(hidden, w_ref[0, :, :], preferred_element_type=jnp.float32)
        y_ref[1, :, :] = jnp.dot(hidden, w_ref[1, :, :], preferred_element_type=jnp.float32)
        self_ref[...] = (
            jnp.dot(hidden, ws_ref[...], preferred_element_type=jnp.float32) + b_ref[...]
        )

    def call(p, sp, w, ws, b):
        h = w.shape[2]
        return pl.pallas_call(
            body,
            out_shape=[
                jax.ShapeDtypeStruct((2, n, h), jnp.float32),
                jax.ShapeDtypeStruct((n, h), jnp.float32),
            ],
        )(p, sp, w, ws, b)

    return call


def _make_final(n):
    def body(p_ref, sp_ref, cc_ref, ms_ref, o_ref):
        h = p_ref[0, :n, :] + p_ref[1, :n, :] + sp_ref[...]
        h2 = h[:, :2]
        hmin = jnp.min(h2)
        cc = cc_ref[...]
        ms = ms_ref[...]
        upper = cc >= ms - 1
        lower = cc == 0
        fill = hmin - 1.0
        o0 = jnp.where(upper, fill, h2[:, 0:1])
        o1 = jnp.where(lower, fill, h2[:, 1:2])
        o_ref[...] = jnp.concatenate([o0, o1], axis=1)

    def call(p, sp, cc, ms):
        return pl.pallas_call(
            body,
            out_shape=jax.ShapeDtypeStruct((n, 2), jnp.float32),
        )(p, sp, cc, ms)

    return call


def _make_edge_agg(h, nchunk, np_rows):
    rpt = np_rows // 16
    mesh = plsc.VectorSubcoreMesh(core_axis_name="c", subcore_axis_name="s")

    @functools.partial(
        pl.kernel,
        mesh=mesh,
        compiler_params=pltpu.CompilerParams(use_tc_tiling_on_sc=False),
        out_type=jax.ShapeDtypeStruct((2, np_rows, h), jnp.float32),
        scratch_types=[
            pltpu.VMEM((nchunk, 128), jnp.int32),
            pltpu.VMEM((nchunk, 128), jnp.int32),
            pltpu.VMEM((128, h), jnp.float32),
            pltpu.VMEM((rpt, h), jnp.float32),
            pltpu.VMEM_SHARED((np_rows, h), jnp.float32),
            pltpu.SemaphoreType.DMA,
        ],
    )
    def k(y_hbm, g_hbm, d_hbm, z_hbm, out_hbm, gi, di, rows, stage, acc, sem):
        c = lax.axis_index("c")
        s = lax.axis_index("s")
        wid = c * 16 + s
        pltpu.sync_copy(z_hbm.at[pl.ds(s * rpt, rpt)], stage)
        pltpu.sync_copy(stage, acc.at[pl.ds(s * rpt, rpt)])
        pltpu.sync_copy(g_hbm.at[wid], gi)
        pltpu.sync_copy(d_hbm.at[wid], di)
        plsc.subcore_barrier()

        def body(j, carry):
            pltpu.async_copy(y_hbm.at[gi.at[j]], rows, sem).wait()
            pltpu.sync_copy(rows, acc.at[di.at[j]], add=True)
            return carry

        lax.fori_loop(0, nchunk, body, 0)
        plsc.subcore_barrier()
        pltpu.sync_copy(acc.at[pl.ds(s * rpt, rpt)], stage)
        pltpu.sync_copy(stage, out_hbm.at[c, pl.ds(s * rpt, rpt)])

    return k


def kernel(x, edge_index, edge_types, cell_types, max_size,
           W1, Ws1, b1, W2, Ws2, b2, W3, Ws3, b3):
    n, _ = x.shape
    e = edge_index.shape[1]
    h = W1.shape[2]
    h3 = 16
    np_rows = ((n + 16 + 127) // 128) * 128

    nchunk = -(-e // (32 * 128))
    epad = 32 * nchunk * 128

    src = edge_index[0]
    dst = edge_index[1]
    g = src + edge_types * n
    g = jnp.concatenate([g, jnp.zeros((epad - e,), jnp.int32)])
    d = jnp.concatenate([dst, jnp.full((epad - e,), np_rows - 1, jnp.int32)])
    g = g.reshape(32, nchunk, 128)
    d = d.reshape(32, nchunk, 128)

    zeros_h = jnp.zeros((np_rows, h), jnp.float32)
    zeros_3 = jnp.zeros((np_rows, h3), jnp.float32)

    W3p = jnp.zeros((2, h, h3), jnp.float32).at[:, :, :2].set(W3)
    Ws3p = jnp.zeros((h, h3), jnp.float32).at[:, :2].set(Ws3)
    b3p = jnp.zeros((1, h3), jnp.float32).at[:, :2].set(b3)

    agg_h = _make_edge_agg(h, nchunk, np_rows)
    agg_3 = _make_edge_agg(h3, nchunk, np_rows)
    dense_mid = _make_dense_mid(n)
    final = _make_final(n)

    y1, s1 = _dense_first(x, W1, Ws1, b1.reshape(1, h))
    p1 = agg_h(y1.reshape(2 * n, h), g, d, zeros_h)
    y2, s2 = dense_mid(p1, s1, W2, Ws2, b2.reshape(1, h))
    p2 = agg_h(y2.reshape(2 * n, h), g, d, zeros_h)
    y3, s3 = dense_mid(p2, s2, W3p, Ws3p, b3p)
    p3 = agg_3(y3.reshape(2 * n, h3), g, d, zeros_3)
    out = final(p3, s3, cell_types[:, 1:2], max_size.reshape(n, 1))
    return out
